# baseline (speedup 1.0000x reference)
import functools

import jax
import jax.numpy as jnp
from jax import lax
from jax.experimental import pallas as pl
from jax.experimental.pallas import tpu as pltpu
from jax.experimental.pallas import tpu_sc as plsc

_E = 64
_K = 2
_D = 2048
_F = 768
_T = 2048
_S = _T * _K
_TM = 256
_NT = _S // _TM
_G = _NT + _E - 1

_NC = 2
_NS = 16
_NW = _NC * _NS


def _gmm_body(ge_ref, tm_ref, lo_ref, hi_ref,
              x_ref, wg_ref, wu_ref, wd_ref, w_ref, o_ref):
    i = pl.program_id(0)
    lo = lo_ref[i]
    hi = hi_ref[i]
    xb = x_ref[...].astype(jnp.bfloat16)
    g = jnp.dot(xb, wg_ref[0].astype(jnp.bfloat16),
                preferred_element_type=jnp.float32)
    u = jnp.dot(xb, wu_ref[0].astype(jnp.bfloat16),
                preferred_element_type=jnp.float32)
    h = (g * jax.lax.logistic(g)) * u
    y = jnp.dot(h.astype(jnp.bfloat16), wd_ref[0].astype(jnp.bfloat16),
                preferred_element_type=jnp.float32)
    y = y * w_ref[:, :1]
    rows = jax.lax.broadcasted_iota(jnp.int32, (_TM, 1), 0)
    mask = (rows >= lo) & (rows < hi)
    first = jnp.logical_or(i == 0, tm_ref[jnp.maximum(i - 1, 0)] != tm_ref[i])
    prev = jnp.where(first, jnp.zeros_like(y), o_ref[...])
    o_ref[...] = jnp.where(mask, y, prev)


def _grouped_ffn(x_sorted, counts, w_bc, w_gate, w_up, w_down):
    offs = jnp.concatenate([jnp.zeros((1,), jnp.int32),
                            jnp.cumsum(counts)[:-1].astype(jnp.int32)])
    t_first = offs // _TM
    t_last = (offs + counts - 1) // _TM
    touched = jnp.where(counts > 0, t_last - t_first + 1, 0).astype(jnp.int32)
    incl = jnp.cumsum(touched)
    pair_off = incl - touched
    total_pairs = incl[-1]

    j = jnp.arange(_G, dtype=jnp.int32)
    ge_raw = jnp.searchsorted(incl, j, side="right").astype(jnp.int32)
    ge_raw = jnp.minimum(ge_raw, _E - 1)
    last_e = jnp.searchsorted(incl, total_pairs - 1, side="right").astype(jnp.int32)
    last_e = jnp.minimum(last_e, _E - 1)
    valid = j < total_pairs
    ge = jnp.where(valid, ge_raw, last_e)
    tm = jnp.where(valid, t_first[ge] + (j - pair_off[ge]), _NT - 1)
    tm = jnp.clip(tm, 0, _NT - 1).astype(jnp.int32)
    base = tm * _TM
    lo = jnp.where(valid, jnp.clip(offs[ge] - base, 0, _TM), 0).astype(jnp.int32)
    hi = jnp.where(valid, jnp.clip(offs[ge] + counts[ge] - base, 0, _TM), 0)
    hi = hi.astype(jnp.int32)

    grid_spec = pltpu.PrefetchScalarGridSpec(
        num_scalar_prefetch=4,
        grid=(_G,),
        in_specs=[
            pl.BlockSpec((_TM, _D), lambda i, ge, tm, lo, hi: (tm[i], 0)),
            pl.BlockSpec((1, _D, _F), lambda i, ge, tm, lo, hi: (ge[i], 0, 0)),
            pl.BlockSpec((1, _D, _F), lambda i, ge, tm, lo, hi: (ge[i], 0, 0)),
            pl.BlockSpec((1, _F, _D), lambda i, ge, tm, lo, hi: (ge[i], 0, 0)),
            pl.BlockSpec((_TM, 128), lambda i, ge, tm, lo, hi: (tm[i], 0)),
        ],
        out_specs=pl.BlockSpec((_TM, _D), lambda i, ge, tm, lo, hi: (tm[i], 0)),
    )
    return pl.pallas_call(
        _gmm_body,
        grid_spec=grid_spec,
        out_shape=jax.ShapeDtypeStruct((_S, _D), jnp.float32),
        compiler_params=pltpu.CompilerParams(
            dimension_semantics=("arbitrary",),
        ),
    )(ge, tm, lo, hi, x_sorted, w_gate, w_up, w_down, w_bc)


def _routing_body(p_ref, pos1_ref, pos2_ref, w1_ref, w2_ref, cnt_ref):
    p = p_ref[...]
    lane = jax.lax.broadcasted_iota(jnp.int32, (_T, _E), 1)
    m1 = jnp.max(p, axis=1, keepdims=True)
    i1 = jnp.min(jnp.where(p == m1, lane, _E), axis=1, keepdims=True)
    oh1 = lane == i1
    pm = jnp.where(oh1, -1.0, p)
    m2 = jnp.max(pm, axis=1, keepdims=True)
    i2 = jnp.min(jnp.where(pm == m2, lane, _E), axis=1, keepdims=True)
    oh2 = lane == i2
    s = m1 + m2
    w1_ref[...] = jnp.broadcast_to(m1 / s, (_T, 128))
    w2_ref[...] = jnp.broadcast_to(m2 / s, (_T, 128))
    ohs = (oh1 | oh2).astype(jnp.float32)
    r = jax.lax.broadcasted_iota(jnp.int32, (_T, _T), 0)
    c = jax.lax.broadcasted_iota(jnp.int32, (_T, _T), 1)
    strict_lt = (c < r).astype(jnp.bfloat16)
    prevc = jnp.dot(strict_lt, ohs.astype(jnp.bfloat16),
                    preferred_element_type=jnp.float32)
    total = jnp.sum(ohs, axis=0, keepdims=True)
    incl = total
    for k in (1, 2, 4, 8, 16, 32):
        incl = incl + jnp.concatenate(
            [jnp.zeros((1, k), jnp.float32), incl[:, :-k]], axis=1)
    offs = incl - total
    base = prevc + offs
    pos1 = jnp.sum(jnp.where(oh1, base, 0.0), axis=1, keepdims=True)
    pos2 = jnp.sum(jnp.where(oh2, base, 0.0), axis=1, keepdims=True)
    pos1_ref[...] = jnp.broadcast_to(pos1, (_T, 128)).astype(jnp.int32)
    pos2_ref[...] = jnp.broadcast_to(pos2, (_T, 128)).astype(jnp.int32)
    cnt_ref[...] = jnp.broadcast_to(total, (8, _E)).astype(jnp.int32)


def _routing(probs):
    return pl.pallas_call(
        _routing_body,
        out_shape=[
            jax.ShapeDtypeStruct((_T, 128), jnp.int32),
            jax.ShapeDtypeStruct((_T, 128), jnp.int32),
            jax.ShapeDtypeStruct((_T, 128), jnp.float32),
            jax.ShapeDtypeStruct((_T, 128), jnp.float32),
            jax.ShapeDtypeStruct((8, _E), jnp.int32),
        ],
    )(probs)


_SC_MESH = plsc.VectorSubcoreMesh(core_axis_name="c", subcore_axis_name="s")


def _dispatch(x, src):
    rows_per_w = _S // _NW
    chunk = 16
    n_chunks = rows_per_w // chunk

    @functools.partial(
        pl.kernel, mesh=_SC_MESH,
        out_type=jax.ShapeDtypeStruct((_S, _D), jnp.float32),
        scratch_types=[
            pltpu.VMEM((rows_per_w,), jnp.int32),
            pltpu.VMEM((2, chunk, _D), jnp.float32),
            pltpu.SemaphoreType.DMA,
            pltpu.SemaphoreType.DMA,
        ],
    )
    def k(x_hbm, src_hbm, out_hbm, idx_v, b_v, s0, s1):
        wid = lax.axis_index("s") * _NC + lax.axis_index("c")
        base = wid * rows_per_w
        pltpu.sync_copy(src_hbm.at[pl.ds(base, rows_per_w)], idx_v)
        sems = (s0, s1)

        def issue(c, par):
            return pltpu.async_copy(
                x_hbm.at[idx_v.at[pl.ds(c * chunk, chunk)]], b_v.at[par],
                sems[par])

        pending = issue(0, 0)
        for c in range(n_chunks):
            par = c % 2
            nxt = issue(c + 1, 1 - par) if c + 1 < n_chunks else None
            pending.wait()
            pltpu.sync_copy(b_v.at[par],
                            out_hbm.at[pl.ds(base + c * chunk, chunk)])
            pending = nxt

    return k(x, src)


def _combine_gather(y2, p0, p1):
    toks_per_w = _T // _NW

    chunk = 8
    n_chunks = toks_per_w // chunk

    @functools.partial(
        pl.kernel, mesh=_SC_MESH,
        out_type=[jax.ShapeDtypeStruct((_T, _D), jnp.float32),
                  jax.ShapeDtypeStruct((_T, _D), jnp.float32)],
        scratch_types=[
            pltpu.VMEM((toks_per_w,), jnp.int32),
            pltpu.VMEM((toks_per_w,), jnp.int32),
            pltpu.VMEM((2, chunk, _D), jnp.float32),
            pltpu.VMEM((2, chunk, _D), jnp.float32),
            pltpu.SemaphoreType.DMA,
            pltpu.SemaphoreType.DMA,
            pltpu.SemaphoreType.DMA,
            pltpu.SemaphoreType.DMA,
        ],
    )
    def k(y_hbm, p0_hbm, p1_hbm, o0_hbm, o1_hbm,
          p0_v, p1_v, b0_v, b1_v, s00, s01, s10, s11):
        wid = lax.axis_index("s") * _NC + lax.axis_index("c")
        base = wid * toks_per_w
        pltpu.sync_copy(p0_hbm.at[pl.ds(base, toks_per_w)], p0_v)
        pltpu.sync_copy(p1_hbm.at[pl.ds(base, toks_per_w)], p1_v)
        sems = ((s00, s10), (s01, s11))

        def issue(c, par):
            sa, sb = sems[par]
            h0 = pltpu.async_copy(
                y_hbm.at[p0_v.at[pl.ds(c * chunk, chunk)]], b0_v.at[par], sa)
            h1 = pltpu.async_copy(
                y_hbm.at[p1_v.at[pl.ds(c * chunk, chunk)]], b1_v.at[par], sb)
            return h0, h1

        pending = issue(0, 0)
        for c in range(n_chunks):
            par = c % 2
            nxt = issue(c + 1, 1 - par) if c + 1 < n_chunks else None
            pending[0].wait()
            pending[1].wait()
            pltpu.sync_copy(b0_v.at[par],
                            o0_hbm.at[pl.ds(base + c * chunk, chunk)])
            pltpu.sync_copy(b1_v.at[par],
                            o1_hbm.at[pl.ds(base + c * chunk, chunk)])
            pending = nxt

    return k(y2, p0, p1)


def kernel(x, router_w, w_gate, w_up, w_down):
    logits = x @ router_w
    probs = jax.nn.softmax(logits.astype(jnp.float32), axis=-1)
    pos1b, pos2b, w1b, w2b, cntb = _routing(probs)
    pos1 = pos1b[:, 0]
    pos2 = pos2b[:, 0]
    counts = cntb[0]
    ar = jnp.arange(_T, dtype=jnp.int32)
    src = (jnp.zeros((_S,), jnp.int32).at[pos1].set(ar).at[pos2].set(ar))
    w_flat = (jnp.zeros((_S,), jnp.float32)
              .at[pos1].set(w1b[:, 0]).at[pos2].set(w2b[:, 0]))
    w_bc = jnp.broadcast_to(w_flat[:, None], (_S, 128))

    x_sorted = _dispatch(x, src)
    y_sorted = _grouped_ffn(x_sorted, counts, w_bc, w_gate, w_up, w_down)
    o0, o1 = _combine_gather(y_sorted, pos1, pos2)
    return (o0 + o1).astype(x.dtype)


# speedup vs baseline: 1.0036x; 1.0036x over previous
import functools

import jax
import jax.numpy as jnp
from jax import lax
from jax.experimental import pallas as pl
from jax.experimental.pallas import tpu as pltpu
from jax.experimental.pallas import tpu_sc as plsc

_E = 64
_K = 2
_D = 2048
_F = 768
_T = 2048
_S = _T * _K
_TM = 256
_NT = _S // _TM
_G = _NT + _E - 1

_NC = 2
_NS = 16
_NW = _NC * _NS


def _gmm_body(ge_ref, tm_ref, lo_ref, hi_ref,
              x_ref, wg_ref, wu_ref, wd_ref, w_ref, o_ref):
    i = pl.program_id(0)
    lo = lo_ref[i]
    hi = hi_ref[i]
    xb = x_ref[...].astype(jnp.bfloat16)
    g = jnp.dot(xb, wg_ref[0].astype(jnp.bfloat16),
                preferred_element_type=jnp.float32)
    u = jnp.dot(xb, wu_ref[0].astype(jnp.bfloat16),
                preferred_element_type=jnp.float32)
    h = (g * jax.lax.logistic(g)) * u
    y = jnp.dot(h.astype(jnp.bfloat16), wd_ref[0].astype(jnp.bfloat16),
                preferred_element_type=jnp.float32)
    y = y * w_ref[:, :1]
    rows = jax.lax.broadcasted_iota(jnp.int32, (_TM, 1), 0)
    mask = (rows >= lo) & (rows < hi)
    first = jnp.logical_or(i == 0, tm_ref[jnp.maximum(i - 1, 0)] != tm_ref[i])
    prev = jnp.where(first, jnp.zeros_like(y), o_ref[...])
    o_ref[...] = jnp.where(mask, y, prev)


def _grouped_ffn(x_sorted, counts, w_bc, w_gate, w_up, w_down):
    offs = jnp.concatenate([jnp.zeros((1,), jnp.int32),
                            jnp.cumsum(counts)[:-1].astype(jnp.int32)])
    t_first = offs // _TM
    t_last = (offs + counts - 1) // _TM
    touched = jnp.where(counts > 0, t_last - t_first + 1, 0).astype(jnp.int32)
    incl = jnp.cumsum(touched)
    pair_off = incl - touched
    total_pairs = incl[-1]

    j = jnp.arange(_G, dtype=jnp.int32)
    ge_raw = jnp.searchsorted(incl, j, side="right").astype(jnp.int32)
    ge_raw = jnp.minimum(ge_raw, _E - 1)
    last_e = jnp.searchsorted(incl, total_pairs - 1, side="right").astype(jnp.int32)
    last_e = jnp.minimum(last_e, _E - 1)
    valid = j < total_pairs
    ge = jnp.where(valid, ge_raw, last_e)
    tm = jnp.where(valid, t_first[ge] + (j - pair_off[ge]), _NT - 1)
    tm = jnp.clip(tm, 0, _NT - 1).astype(jnp.int32)
    base = tm * _TM
    lo = jnp.where(valid, jnp.clip(offs[ge] - base, 0, _TM), 0).astype(jnp.int32)
    hi = jnp.where(valid, jnp.clip(offs[ge] + counts[ge] - base, 0, _TM), 0)
    hi = hi.astype(jnp.int32)

    grid_spec = pltpu.PrefetchScalarGridSpec(
        num_scalar_prefetch=4,
        grid=(_G,),
        in_specs=[
            pl.BlockSpec((_TM, _D), lambda i, ge, tm, lo, hi: (tm[i], 0)),
            pl.BlockSpec((1, _D, _F), lambda i, ge, tm, lo, hi: (ge[i], 0, 0)),
            pl.BlockSpec((1, _D, _F), lambda i, ge, tm, lo, hi: (ge[i], 0, 0)),
            pl.BlockSpec((1, _F, _D), lambda i, ge, tm, lo, hi: (ge[i], 0, 0)),
            pl.BlockSpec((_TM, 128), lambda i, ge, tm, lo, hi: (tm[i], 0)),
        ],
        out_specs=pl.BlockSpec((_TM, _D), lambda i, ge, tm, lo, hi: (tm[i], 0)),
    )
    return pl.pallas_call(
        _gmm_body,
        grid_spec=grid_spec,
        out_shape=jax.ShapeDtypeStruct((_S, _D), jnp.float32),
        compiler_params=pltpu.CompilerParams(
            dimension_semantics=("arbitrary",),
        ),
    )(ge, tm, lo, hi, x_sorted, w_gate, w_up, w_down, w_bc)


def _routing_body(p_ref, pos1_ref, pos2_ref, w1_ref, w2_ref, cnt_ref):
    p = p_ref[...]
    lane = jax.lax.broadcasted_iota(jnp.int32, (_T, _E), 1)
    m1 = jnp.max(p, axis=1, keepdims=True)
    i1 = jnp.min(jnp.where(p == m1, lane, _E), axis=1, keepdims=True)
    oh1 = lane == i1
    pm = jnp.where(oh1, -1.0, p)
    m2 = jnp.max(pm, axis=1, keepdims=True)
    i2 = jnp.min(jnp.where(pm == m2, lane, _E), axis=1, keepdims=True)
    oh2 = lane == i2
    s = m1 + m2
    w1_ref[...] = jnp.broadcast_to(m1 / s, (_T, 128))
    w2_ref[...] = jnp.broadcast_to(m2 / s, (_T, 128))
    ohs = (oh1 | oh2).astype(jnp.float32)
    r = jax.lax.broadcasted_iota(jnp.int32, (_T, _T), 0)
    c = jax.lax.broadcasted_iota(jnp.int32, (_T, _T), 1)
    strict_lt = (c < r).astype(jnp.bfloat16)
    prevc = jnp.dot(strict_lt, ohs.astype(jnp.bfloat16),
                    preferred_element_type=jnp.float32)
    total = jnp.sum(ohs, axis=0, keepdims=True)
    incl = total
    for k in (1, 2, 4, 8, 16, 32):
        incl = incl + jnp.concatenate(
            [jnp.zeros((1, k), jnp.float32), incl[:, :-k]], axis=1)
    offs = incl - total
    base = prevc + offs
    pos1 = jnp.sum(jnp.where(oh1, base, 0.0), axis=1, keepdims=True)
    pos2 = jnp.sum(jnp.where(oh2, base, 0.0), axis=1, keepdims=True)
    pos1_ref[...] = jnp.broadcast_to(pos1, (_T, 128)).astype(jnp.int32)
    pos2_ref[...] = jnp.broadcast_to(pos2, (_T, 128)).astype(jnp.int32)
    cnt_ref[...] = jnp.broadcast_to(total, (8, _E)).astype(jnp.int32)


def _routing(probs):
    return pl.pallas_call(
        _routing_body,
        out_shape=[
            jax.ShapeDtypeStruct((_T, 128), jnp.int32),
            jax.ShapeDtypeStruct((_T, 128), jnp.int32),
            jax.ShapeDtypeStruct((_T, 128), jnp.float32),
            jax.ShapeDtypeStruct((_T, 128), jnp.float32),
            jax.ShapeDtypeStruct((8, _E), jnp.int32),
        ],
    )(probs)


_SC_MESH = plsc.VectorSubcoreMesh(core_axis_name="c", subcore_axis_name="s")


def _dispatch(x, src):
    rows_per_w = _S // _NW
    chunk = 32

    @functools.partial(
        pl.kernel, mesh=_SC_MESH,
        out_type=jax.ShapeDtypeStruct((_S, _D), jnp.float32),
        scratch_types=[
            pltpu.VMEM((chunk,), jnp.int32),
            pltpu.VMEM((chunk, _D), jnp.float32),
            pltpu.SemaphoreType.DMA,
        ],
    )
    def k(x_hbm, src_hbm, out_hbm, idx_v, rows_v, sem):
        wid = lax.axis_index("s") * _NC + lax.axis_index("c")
        base = wid * rows_per_w

        @pl.loop(0, rows_per_w // chunk)
        def _(c):
            sb = base + c * chunk
            pltpu.sync_copy(src_hbm.at[pl.ds(sb, chunk)], idx_v)
            pltpu.async_copy(x_hbm.at[idx_v], rows_v, sem).wait()
            pltpu.sync_copy(rows_v, out_hbm.at[pl.ds(sb, chunk)])

    return k(x, src)


def _combine_gather(y2, p0, p1):
    toks_per_w = _T // _NW

    chunk = 8
    n_chunks = toks_per_w // chunk

    @functools.partial(
        pl.kernel, mesh=_SC_MESH,
        out_type=[jax.ShapeDtypeStruct((_T, _D), jnp.float32),
                  jax.ShapeDtypeStruct((_T, _D), jnp.float32)],
        scratch_types=[
            pltpu.VMEM((toks_per_w,), jnp.int32),
            pltpu.VMEM((toks_per_w,), jnp.int32),
            pltpu.VMEM((2, chunk, _D), jnp.float32),
            pltpu.VMEM((2, chunk, _D), jnp.float32),
            pltpu.SemaphoreType.DMA,
            pltpu.SemaphoreType.DMA,
            pltpu.SemaphoreType.DMA,
            pltpu.SemaphoreType.DMA,
        ],
    )
    def k(y_hbm, p0_hbm, p1_hbm, o0_hbm, o1_hbm,
          p0_v, p1_v, b0_v, b1_v, s00, s01, s10, s11):
        wid = lax.axis_index("s") * _NC + lax.axis_index("c")
        base = wid * toks_per_w
        pltpu.sync_copy(p0_hbm.at[pl.ds(base, toks_per_w)], p0_v)
        pltpu.sync_copy(p1_hbm.at[pl.ds(base, toks_per_w)], p1_v)
        sems = ((s00, s10), (s01, s11))

        def issue(c, par):
            sa, sb = sems[par]
            h0 = pltpu.async_copy(
                y_hbm.at[p0_v.at[pl.ds(c * chunk, chunk)]], b0_v.at[par], sa)
            h1 = pltpu.async_copy(
                y_hbm.at[p1_v.at[pl.ds(c * chunk, chunk)]], b1_v.at[par], sb)
            return h0, h1

        pending = issue(0, 0)
        for c in range(n_chunks):
            par = c % 2
            nxt = issue(c + 1, 1 - par) if c + 1 < n_chunks else None
            pending[0].wait()
            pending[1].wait()
            pltpu.sync_copy(b0_v.at[par],
                            o0_hbm.at[pl.ds(base + c * chunk, chunk)])
            pltpu.sync_copy(b1_v.at[par],
                            o1_hbm.at[pl.ds(base + c * chunk, chunk)])
            pending = nxt

    return k(y2, p0, p1)


def kernel(x, router_w, w_gate, w_up, w_down):
    logits = x @ router_w
    probs = jax.nn.softmax(logits.astype(jnp.float32), axis=-1)
    pos1b, pos2b, w1b, w2b, cntb = _routing(probs)
    pos1 = pos1b[:, 0]
    pos2 = pos2b[:, 0]
    counts = cntb[0]
    ar = jnp.arange(_T, dtype=jnp.int32)
    src = (jnp.zeros((_S,), jnp.int32).at[pos1].set(ar).at[pos2].set(ar))
    w_flat = (jnp.zeros((_S,), jnp.float32)
              .at[pos1].set(w1b[:, 0]).at[pos2].set(w2b[:, 0]))
    w_bc = jnp.broadcast_to(w_flat[:, None], (_S, 128))

    x_sorted = _dispatch(x, src)
    y_sorted = _grouped_ffn(x_sorted, counts, w_bc, w_gate, w_up, w_down)
    o0, o1 = _combine_gather(y_sorted, pos1, pos2)
    return (o0 + o1).astype(x.dtype)


# speedup vs baseline: 1.0045x; 1.0009x over previous
import functools

import jax
import jax.numpy as jnp
from jax import lax
from jax.experimental import pallas as pl
from jax.experimental.pallas import tpu as pltpu
from jax.experimental.pallas import tpu_sc as plsc

_E = 64
_K = 2
_D = 2048
_F = 768
_T = 2048
_S = _T * _K
_TM = 256
_NT = _S // _TM
_G = _NT + _E - 1

_NC = 2
_NS = 16
_NW = _NC * _NS


def _gmm_body(ge_ref, tm_ref, lo_ref, hi_ref,
              x_ref, wg_ref, wu_ref, wd_ref, w_ref, o_ref):
    i = pl.program_id(0)
    lo = lo_ref[i]
    hi = hi_ref[i]
    xb = x_ref[...].astype(jnp.bfloat16)
    g = jnp.dot(xb, wg_ref[0].astype(jnp.bfloat16),
                preferred_element_type=jnp.float32)
    u = jnp.dot(xb, wu_ref[0].astype(jnp.bfloat16),
                preferred_element_type=jnp.float32)
    h = (g * jax.lax.logistic(g)) * u
    y = jnp.dot(h.astype(jnp.bfloat16), wd_ref[0].astype(jnp.bfloat16),
                preferred_element_type=jnp.float32)
    y = y * w_ref[:, :1]
    rows = jax.lax.broadcasted_iota(jnp.int32, (_TM, 1), 0)
    mask = (rows >= lo) & (rows < hi)
    first = jnp.logical_or(i == 0, tm_ref[jnp.maximum(i - 1, 0)] != tm_ref[i])
    prev = jnp.where(first, jnp.zeros_like(y), o_ref[...])
    o_ref[...] = jnp.where(mask, y, prev)


def _grouped_ffn(x_sorted, counts, w_bc, w_gate, w_up, w_down):
    offs = jnp.concatenate([jnp.zeros((1,), jnp.int32),
                            jnp.cumsum(counts)[:-1].astype(jnp.int32)])
    t_first = offs // _TM
    t_last = (offs + counts - 1) // _TM
    touched = jnp.where(counts > 0, t_last - t_first + 1, 0).astype(jnp.int32)
    incl = jnp.cumsum(touched)
    pair_off = incl - touched
    total_pairs = incl[-1]

    j = jnp.arange(_G, dtype=jnp.int32)
    ge_raw = jnp.searchsorted(incl, j, side="right").astype(jnp.int32)
    ge_raw = jnp.minimum(ge_raw, _E - 1)
    last_e = jnp.searchsorted(incl, total_pairs - 1, side="right").astype(jnp.int32)
    last_e = jnp.minimum(last_e, _E - 1)
    valid = j < total_pairs
    ge = jnp.where(valid, ge_raw, last_e)
    tm = jnp.where(valid, t_first[ge] + (j - pair_off[ge]), _NT - 1)
    tm = jnp.clip(tm, 0, _NT - 1).astype(jnp.int32)
    base = tm * _TM
    lo = jnp.where(valid, jnp.clip(offs[ge] - base, 0, _TM), 0).astype(jnp.int32)
    hi = jnp.where(valid, jnp.clip(offs[ge] + counts[ge] - base, 0, _TM), 0)
    hi = hi.astype(jnp.int32)

    grid_spec = pltpu.PrefetchScalarGridSpec(
        num_scalar_prefetch=4,
        grid=(_G,),
        in_specs=[
            pl.BlockSpec((_TM, _D), lambda i, ge, tm, lo, hi: (tm[i], 0)),
            pl.BlockSpec((1, _D, _F), lambda i, ge, tm, lo, hi: (ge[i], 0, 0)),
            pl.BlockSpec((1, _D, _F), lambda i, ge, tm, lo, hi: (ge[i], 0, 0)),
            pl.BlockSpec((1, _F, _D), lambda i, ge, tm, lo, hi: (ge[i], 0, 0)),
            pl.BlockSpec((_TM, 128), lambda i, ge, tm, lo, hi: (tm[i], 0)),
        ],
        out_specs=pl.BlockSpec((_TM, _D), lambda i, ge, tm, lo, hi: (tm[i], 0)),
    )
    return pl.pallas_call(
        _gmm_body,
        grid_spec=grid_spec,
        out_shape=jax.ShapeDtypeStruct((_S, _D), jnp.float32),
        compiler_params=pltpu.CompilerParams(
            dimension_semantics=("arbitrary",),
        ),
    )(ge, tm, lo, hi, x_sorted, w_gate, w_up, w_down, w_bc)


def _routing_body(p_ref, pos1_ref, pos2_ref, w1_ref, w2_ref, cnt_ref):
    p = p_ref[...]
    lane = jax.lax.broadcasted_iota(jnp.int32, (_T, _E), 1)
    m1 = jnp.max(p, axis=1, keepdims=True)
    i1 = jnp.min(jnp.where(p == m1, lane, _E), axis=1, keepdims=True)
    oh1 = lane == i1
    pm = jnp.where(oh1, -1.0, p)
    m2 = jnp.max(pm, axis=1, keepdims=True)
    i2 = jnp.min(jnp.where(pm == m2, lane, _E), axis=1, keepdims=True)
    oh2 = lane == i2
    s = m1 + m2
    w1_ref[...] = jnp.broadcast_to(m1 / s, (_T, 128))
    w2_ref[...] = jnp.broadcast_to(m2 / s, (_T, 128))
    ohs = (oh1 | oh2).astype(jnp.float32)
    r = jax.lax.broadcasted_iota(jnp.int32, (_T, _T), 0)
    c = jax.lax.broadcasted_iota(jnp.int32, (_T, _T), 1)
    strict_lt = (c < r).astype(jnp.bfloat16)
    prevc = jnp.dot(strict_lt, ohs.astype(jnp.bfloat16),
                    preferred_element_type=jnp.float32)
    total = jnp.sum(ohs, axis=0, keepdims=True)
    incl = total
    for k in (1, 2, 4, 8, 16, 32):
        incl = incl + jnp.concatenate(
            [jnp.zeros((1, k), jnp.float32), incl[:, :-k]], axis=1)
    offs = incl - total
    base = prevc + offs
    pos1 = jnp.sum(jnp.where(oh1, base, 0.0), axis=1, keepdims=True)
    pos2 = jnp.sum(jnp.where(oh2, base, 0.0), axis=1, keepdims=True)
    pos1_ref[...] = jnp.broadcast_to(pos1, (_T, 128)).astype(jnp.int32)
    pos2_ref[...] = jnp.broadcast_to(pos2, (_T, 128)).astype(jnp.int32)
    cnt_ref[...] = jnp.broadcast_to(total, (8, _E)).astype(jnp.int32)


def _routing(probs):
    return pl.pallas_call(
        _routing_body,
        out_shape=[
            jax.ShapeDtypeStruct((_T, 128), jnp.int32),
            jax.ShapeDtypeStruct((_T, 128), jnp.int32),
            jax.ShapeDtypeStruct((_T, 128), jnp.float32),
            jax.ShapeDtypeStruct((_T, 128), jnp.float32),
            jax.ShapeDtypeStruct((8, _E), jnp.int32),
        ],
    )(probs)


def _sc_mesh():
    return plsc.VectorSubcoreMesh(core_axis_name="c", subcore_axis_name="s")


def _dispatch(x, src):
    rows_per_w = _S // _NW
    chunk = 32

    @functools.partial(
        pl.kernel, mesh=_sc_mesh(),
        out_type=jax.ShapeDtypeStruct((_S, _D), jnp.float32),
        scratch_types=[
            pltpu.VMEM((chunk,), jnp.int32),
            pltpu.VMEM((chunk, _D), jnp.float32),
            pltpu.SemaphoreType.DMA,
        ],
    )
    def k(x_hbm, src_hbm, out_hbm, idx_v, rows_v, sem):
        wid = lax.axis_index("s") * _NC + lax.axis_index("c")
        base = wid * rows_per_w

        @pl.loop(0, rows_per_w // chunk)
        def _(c):
            sb = base + c * chunk
            pltpu.sync_copy(src_hbm.at[pl.ds(sb, chunk)], idx_v)
            pltpu.async_copy(x_hbm.at[idx_v], rows_v, sem).wait()
            pltpu.sync_copy(rows_v, out_hbm.at[pl.ds(sb, chunk)])

    return k(x, src)


def _combine_gather(y2, p0, p1):
    toks_per_w = _T // _NW

    chunk = 8
    n_chunks = toks_per_w // chunk

    @functools.partial(
        pl.kernel, mesh=_sc_mesh(),
        out_type=[jax.ShapeDtypeStruct((_T, _D), jnp.float32),
                  jax.ShapeDtypeStruct((_T, _D), jnp.float32)],
        scratch_types=[
            pltpu.VMEM((toks_per_w,), jnp.int32),
            pltpu.VMEM((toks_per_w,), jnp.int32),
            pltpu.VMEM((2, chunk, _D), jnp.float32),
            pltpu.VMEM((2, chunk, _D), jnp.float32),
            pltpu.SemaphoreType.DMA,
            pltpu.SemaphoreType.DMA,
            pltpu.SemaphoreType.DMA,
            pltpu.SemaphoreType.DMA,
        ],
    )
    def k(y_hbm, p0_hbm, p1_hbm, o0_hbm, o1_hbm,
          p0_v, p1_v, b0_v, b1_v, s00, s01, s10, s11):
        wid = lax.axis_index("s") * _NC + lax.axis_index("c")
        base = wid * toks_per_w
        pltpu.sync_copy(p0_hbm.at[pl.ds(base, toks_per_w)], p0_v)
        pltpu.sync_copy(p1_hbm.at[pl.ds(base, toks_per_w)], p1_v)
        sems = ((s00, s10), (s01, s11))

        def issue(c, par):
            sa, sb = sems[par]
            h0 = pltpu.async_copy(
                y_hbm.at[p0_v.at[pl.ds(c * chunk, chunk)]], b0_v.at[par], sa)
            h1 = pltpu.async_copy(
                y_hbm.at[p1_v.at[pl.ds(c * chunk, chunk)]], b1_v.at[par], sb)
            return h0, h1

        pending = issue(0, 0)
        for c in range(n_chunks):
            par = c % 2
            nxt = issue(c + 1, 1 - par) if c + 1 < n_chunks else None
            pending[0].wait()
            pending[1].wait()
            pltpu.sync_copy(b0_v.at[par],
                            o0_hbm.at[pl.ds(base + c * chunk, chunk)])
            pltpu.sync_copy(b1_v.at[par],
                            o1_hbm.at[pl.ds(base + c * chunk, chunk)])
            pending = nxt

    return k(y2, p0, p1)


def kernel(x, router_w, w_gate, w_up, w_down):
    logits = x @ router_w
    probs = jax.nn.softmax(logits.astype(jnp.float32), axis=-1)
    pos1b, pos2b, w1b, w2b, cntb = _routing(probs)
    pos1 = pos1b[:, 0]
    pos2 = pos2b[:, 0]
    counts = cntb[0]
    ar = jnp.arange(_T, dtype=jnp.int32)
    src = (jnp.zeros((_S,), jnp.int32).at[pos1].set(ar).at[pos2].set(ar))
    w_flat = (jnp.zeros((_S,), jnp.float32)
              .at[pos1].set(w1b[:, 0]).at[pos2].set(w2b[:, 0]))
    w_bc = jnp.broadcast_to(w_flat[:, None], (_S, 128))

    x_sorted = _dispatch(x, src)
    y_sorted = _grouped_ffn(x_sorted, counts, w_bc, w_gate, w_up, w_down)
    o0, o1 = _combine_gather(y_sorted, pos1, pos2)
    return (o0 + o1).astype(x.dtype)
